# baseline (device time: 9585 ns/iter reference)
import jax
import jax.numpy as jnp
from jax import lax
from jax.experimental import pallas as pl
from jax.experimental.pallas import tpu as pltpu

_CHUNK = 128


def kernel(x, dy, gamma):
    m, d = x.shape
    half = m // 2
    n_chunks = half // _CHUNK

    off = lax.axis_index("y") * half
    stacked = jnp.stack([
        lax.dynamic_slice(x, (off, 0), (half, d)),
        lax.dynamic_slice(dy, (off, 0), (half, d)),
    ])

    def body(s_hbm, out_ref, xbuf, dybuf, comm_ref,
             xsems, dysems, send_sems, recv_sems):
        my_x = lax.axis_index("x")
        my_y = lax.axis_index("y")
        peers = (
            (1 - my_x, my_y),
            (my_x, 1 - my_y),
            (1 - my_x, 1 - my_y),
        )

        barrier_sem = pltpu.get_barrier_semaphore()
        for nbr in peers:
            pl.semaphore_signal(
                barrier_sem, inc=1,
                device_id=nbr, device_id_type=pl.DeviceIdType.MESH,
            )

        def chunk_copies(i, slot):
            cp_x = pltpu.make_async_copy(
                s_hbm.at[0, pl.ds(i * _CHUNK, _CHUNK)],
                xbuf.at[slot], xsems.at[slot])
            cp_dy = pltpu.make_async_copy(
                s_hbm.at[1, pl.ds(i * _CHUNK, _CHUNK)],
                dybuf.at[slot], dysems.at[slot])
            return cp_x, cp_dy

        first = chunk_copies(0, 0)
        first[0].start()
        first[1].start()

        inv_d = jnp.float32(1.0 / d)
        acc_dg = jnp.zeros((d,), jnp.float32)
        acc_db = jnp.zeros((d,), jnp.float32)
        for i in range(n_chunks):
            slot = i % 2
            if i + 1 < n_chunks:
                nxt = chunk_copies(i + 1, (i + 1) % 2)
                nxt[0].start()
                nxt[1].start()
            wait_cp = chunk_copies(i, slot)
            wait_cp[0].wait()
            wait_cp[1].wait()
            xv = xbuf[slot]
            dyv = dybuf[slot]
            mu = jnp.sum(xv, axis=1, keepdims=True) * inv_d
            mean2 = jnp.sum(xv * xv, axis=1, keepdims=True) * inv_d
            rstd = lax.rsqrt(mean2 - mu * mu + 1e-5)
            acc_dg = acc_dg + jnp.sum(dyv * ((xv - mu) * rstd), axis=0)
            acc_db = acc_db + jnp.sum(dyv, axis=0)

        comm_ref[0, 0, :] = acc_dg
        comm_ref[0, 1, :] = acc_db

        pl.semaphore_wait(barrier_sem, 3)

        rdmas = []
        for i, nbr in enumerate(peers):
            rdma = pltpu.make_async_remote_copy(
                src_ref=comm_ref.at[0], dst_ref=comm_ref.at[i + 1],
                send_sem=send_sems.at[i], recv_sem=recv_sems.at[i],
                device_id=nbr, device_id_type=pl.DeviceIdType.MESH,
            )
            rdma.start()
            rdmas.append(rdma)
        for rdma in rdmas:
            rdma.wait_recv()
        out_ref[:, :] = (comm_ref[0] + comm_ref[1]) + (comm_ref[2] + comm_ref[3])
        for rdma in rdmas:
            rdma.wait_send()

    return pl.pallas_call(
        body,
        out_shape=jax.ShapeDtypeStruct((2, d), jnp.float32),
        in_specs=[pl.BlockSpec(memory_space=pl.ANY)],
        out_specs=pl.BlockSpec(memory_space=pltpu.VMEM),
        scratch_shapes=[
            pltpu.VMEM((2, _CHUNK, d), jnp.float32),
            pltpu.VMEM((2, _CHUNK, d), jnp.float32),
            pltpu.VMEM((4, 2, d), jnp.float32),
            pltpu.SemaphoreType.DMA((2,)),
            pltpu.SemaphoreType.DMA((2,)),
            pltpu.SemaphoreType.DMA((3,)),
            pltpu.SemaphoreType.DMA((3,)),
        ],
        compiler_params=pltpu.CompilerParams(collective_id=0),
    )(stacked)


# device time: 9161 ns/iter; 1.0463x vs baseline; 1.0463x over previous
import jax
import jax.numpy as jnp
from jax import lax
from jax.experimental import pallas as pl
from jax.experimental.pallas import tpu as pltpu


def kernel(x, dy, gamma):
    m, d = x.shape
    half = m // 2

    off = lax.axis_index("y") * half
    stacked = jnp.stack([
        lax.dynamic_slice(x, (off, 0), (half, d)),
        lax.dynamic_slice(dy, (off, 0), (half, d)),
    ]).astype(jnp.bfloat16)

    def body(s_ref, out_ref, comm_ref, send_sems, recv_sems):
        my_x = lax.axis_index("x")
        my_y = lax.axis_index("y")
        peers = (
            (1 - my_x, my_y),
            (my_x, 1 - my_y),
            (1 - my_x, 1 - my_y),
        )

        barrier_sem = pltpu.get_barrier_semaphore()
        for nbr in peers:
            pl.semaphore_signal(
                barrier_sem, inc=1,
                device_id=nbr, device_id_type=pl.DeviceIdType.MESH,
            )

        xv = s_ref[0].astype(jnp.float32)
        dyv = s_ref[1].astype(jnp.float32)
        inv_d = jnp.float32(1.0 / d)
        mu = jnp.sum(xv, axis=1, keepdims=True) * inv_d
        mean2 = jnp.sum(xv * xv, axis=1, keepdims=True) * inv_d
        rstd = lax.rsqrt(mean2 - mu * mu + 1e-5)
        comm_ref[0, 0, :] = jnp.sum(dyv * ((xv - mu) * rstd), axis=0)
        comm_ref[0, 1, :] = jnp.sum(dyv, axis=0)

        pl.semaphore_wait(barrier_sem, 3)

        rdmas = []
        for i, nbr in enumerate(peers):
            rdma = pltpu.make_async_remote_copy(
                src_ref=comm_ref.at[0], dst_ref=comm_ref.at[i + 1],
                send_sem=send_sems.at[i], recv_sem=recv_sems.at[i],
                device_id=nbr, device_id_type=pl.DeviceIdType.MESH,
            )
            rdma.start()
            rdmas.append(rdma)
        for rdma in rdmas:
            rdma.wait_recv()
        out_ref[:, :] = (comm_ref[0] + comm_ref[1]) + (comm_ref[2] + comm_ref[3])
        for rdma in rdmas:
            rdma.wait_send()

    return pl.pallas_call(
        body,
        out_shape=jax.ShapeDtypeStruct((2, d), jnp.float32),
        in_specs=[pl.BlockSpec(memory_space=pltpu.VMEM)],
        out_specs=pl.BlockSpec(memory_space=pltpu.VMEM),
        scratch_shapes=[
            pltpu.VMEM((4, 2, d), jnp.float32),
            pltpu.SemaphoreType.DMA((3,)),
            pltpu.SemaphoreType.DMA((3,)),
        ],
        compiler_params=pltpu.CompilerParams(collective_id=0),
    )(stacked)


# device time: 9132 ns/iter; 1.0496x vs baseline; 1.0032x over previous
import jax
import jax.numpy as jnp
from jax import lax
from jax.experimental import pallas as pl
from jax.experimental.pallas import tpu as pltpu


def kernel(x, dy, gamma):
    m, d = x.shape
    half = m // 2

    off = lax.axis_index("y") * half
    stacked = jnp.stack([
        lax.dynamic_slice(x, (off, 0), (half, d)),
        lax.dynamic_slice(dy, (off, 0), (half, d)),
    ])

    def body(s_ref, out_ref, comm_ref, send_sems, recv_sems):
        my_x = lax.axis_index("x")
        my_y = lax.axis_index("y")
        peers = (
            (1 - my_x, my_y),
            (my_x, 1 - my_y),
            (1 - my_x, 1 - my_y),
        )

        barrier_sem = pltpu.get_barrier_semaphore()
        for nbr in peers:
            pl.semaphore_signal(
                barrier_sem, inc=1,
                device_id=nbr, device_id_type=pl.DeviceIdType.MESH,
            )

        xv = s_ref[0]
        dyv = s_ref[1]
        inv_d = jnp.float32(1.0 / d)
        mu = jnp.sum(xv, axis=1, keepdims=True) * inv_d
        mean2 = jnp.sum(xv * xv, axis=1, keepdims=True) * inv_d
        rstd = lax.rsqrt(mean2 - mu * mu + 1e-5)
        comm_ref[0, 0, :] = jnp.sum(dyv * ((xv - mu) * rstd), axis=0)
        comm_ref[0, 1, :] = jnp.sum(dyv, axis=0)

        pl.semaphore_wait(barrier_sem, 3)

        rdmas = []
        for i, nbr in enumerate(peers):
            rdma = pltpu.make_async_remote_copy(
                src_ref=comm_ref.at[0], dst_ref=comm_ref.at[i + 1],
                send_sem=send_sems.at[i], recv_sem=recv_sems.at[i],
                device_id=nbr, device_id_type=pl.DeviceIdType.MESH,
            )
            rdma.start()
            rdmas.append(rdma)
        for rdma in rdmas:
            rdma.wait_recv()
        out_ref[:, :] = (comm_ref[0] + comm_ref[1]) + (comm_ref[2] + comm_ref[3])
        for rdma in rdmas:
            rdma.wait_send()

    return pl.pallas_call(
        body,
        out_shape=jax.ShapeDtypeStruct((2, d), jnp.float32),
        in_specs=[pl.BlockSpec(memory_space=pltpu.VMEM)],
        out_specs=pl.BlockSpec(memory_space=pltpu.VMEM),
        scratch_shapes=[
            pltpu.VMEM((4, 2, d), jnp.float32),
            pltpu.SemaphoreType.DMA((3,)),
            pltpu.SemaphoreType.DMA((3,)),
        ],
        compiler_params=pltpu.CompilerParams(collective_id=0),
    )(stacked)
